# baseline (speedup 1.0000x reference)
import functools

import jax
import jax.numpy as jnp
from jax import lax
from jax.experimental import pallas as pl
from jax.experimental.pallas import tpu as pltpu
from jax.experimental.pallas import tpu_sc as plsc

_B = 4096
_IN = 1000
_H = 1000
_D = 256
_K = 8192
_EPS = 1e-5
_BB = 512
_NBLK = _B // _BB
_KC = 2048
_NKC = _K // _KC
_KS = 512


def _mm_bias_body(x_ref, w_ref, b_ref, o_ref, s_ref):
    i = pl.program_id(0)
    h = jnp.dot(x_ref[...], w_ref[...], preferred_element_type=jnp.float32)
    h = h + b_ref[...]
    o_ref[...] = h
    part = jnp.sum(h, axis=0, keepdims=True)

    @pl.when(i == 0)
    def _():
        s_ref[...] = part

    @pl.when(i > 0)
    def _():
        s_ref[...] = s_ref[...] + part


def _mm_bias(x, w, b, n_in, n_out):
    return pl.pallas_call(
        _mm_bias_body,
        grid=(_NBLK,),
        in_specs=[
            pl.BlockSpec((_BB, n_in), lambda i: (i, 0)),
            pl.BlockSpec((n_in, n_out), lambda i: (0, 0)),
            pl.BlockSpec((1, n_out), lambda i: (0, 0)),
        ],
        out_specs=[
            pl.BlockSpec((_BB, n_out), lambda i: (i, 0)),
            pl.BlockSpec((1, n_out), lambda i: (0, 0)),
        ],
        out_shape=[
            jax.ShapeDtypeStruct((_B, n_out), jnp.float32),
            jax.ShapeDtypeStruct((1, n_out), jnp.float32),
        ],
    )(x, w, b.reshape(1, n_out))


def _varsum_body(h_ref, s_ref, v_ref):
    i = pl.program_id(0)
    mu = s_ref[...] / float(_B)
    dev = h_ref[...] - mu
    part = jnp.sum(dev * dev, axis=0, keepdims=True)

    @pl.when(i == 0)
    def _():
        v_ref[...] = part

    @pl.when(i > 0)
    def _():
        v_ref[...] = v_ref[...] + part


def _varsum(h, colsum, n_out):
    return pl.pallas_call(
        _varsum_body,
        grid=(_NBLK,),
        in_specs=[
            pl.BlockSpec((_BB, n_out), lambda i: (i, 0)),
            pl.BlockSpec((1, n_out), lambda i: (0, 0)),
        ],
        out_specs=pl.BlockSpec((1, n_out), lambda i: (0, 0)),
        out_shape=jax.ShapeDtypeStruct((1, n_out), jnp.float32),
    )(h, colsum)


def _enc2_vq_body(h_ref, s_ref, v_ref, g_ref, bb_ref, w2_ref, b2_ref, cb_ref,
                  idx_ref, ds_ref):
    i = pl.program_id(0)
    mu = s_ref[...] / float(_B)
    var = v_ref[...] / float(_B)
    hn = (h_ref[...] - mu) / jnp.sqrt(var + _EPS) * g_ref[...] + bb_ref[...]
    a = jnp.maximum(hn, 0.0)
    z = jnp.dot(a, w2_ref[...], preferred_element_type=jnp.float32) + b2_ref[...]

    z2 = jnp.sum(z * z, axis=1, keepdims=True)
    zz = z + z
    best_d = jnp.full((_BB,), jnp.inf, jnp.float32)
    best_i = jnp.zeros((_BB,), jnp.int32)
    iot = lax.broadcasted_iota(jnp.int32, (_BB, _KS), 1)
    for c in range(_NKC):
        cc = cb_ref[pl.ds(c * _KC, _KC), :]
        c2 = jnp.sum(cc * cc, axis=1)[None, :]
        zc2 = lax.dot_general(zz, cc, (((1,), (1,)), ((), ())),
                              preferred_element_type=jnp.float32)
        for t in range(_KC // _KS):
            lo = t * _KS
            d = (z2 + c2[:, lo:lo + _KS]) - zc2[:, lo:lo + _KS]
            tmin = jnp.min(d, axis=1)
            targ = jnp.min(jnp.where(d == tmin[:, None], iot, _KS), axis=1)
            upd = tmin < best_d
            best_i = jnp.where(upd, targ + (c * _KC + lo), best_i)
            best_d = jnp.where(upd, tmin, best_d)

    idx_ref[...] = best_i.reshape(1, 1, _BB)

    lane = lax.broadcasted_iota(jnp.int32, (1, 128), 1)
    dpart = jnp.where(lane == 0, jnp.sum(best_d), 0.0)

    @pl.when(i == 0)
    def _():
        ds_ref[...] = dpart

    @pl.when(i > 0)
    def _():
        ds_ref[...] = ds_ref[...] + dpart


def _enc2_vq(h, colsum, varsum, g, b, w2, b2, codebook):
    return pl.pallas_call(
        _enc2_vq_body,
        grid=(_NBLK,),
        in_specs=[
            pl.BlockSpec((_BB, _H), lambda i: (i, 0)),
            pl.BlockSpec((1, _H), lambda i: (0, 0)),
            pl.BlockSpec((1, _H), lambda i: (0, 0)),
            pl.BlockSpec((1, _H), lambda i: (0, 0)),
            pl.BlockSpec((1, _H), lambda i: (0, 0)),
            pl.BlockSpec((_H, _D), lambda i: (0, 0)),
            pl.BlockSpec((1, _D), lambda i: (0, 0)),
            pl.BlockSpec((_K, _D), lambda i: (0, 0)),
        ],
        out_specs=[
            pl.BlockSpec((1, 1, _BB), lambda i: (i, 0, 0)),
            pl.BlockSpec((1, 128), lambda i: (0, 0)),
        ],
        out_shape=[
            jax.ShapeDtypeStruct((_NBLK, 1, _BB), jnp.int32),
            jax.ShapeDtypeStruct((1, 128), jnp.float32),
        ],
    )(h, colsum, varsum, g.reshape(1, _H), b.reshape(1, _H), w2,
      b2.reshape(1, _D), codebook)


def _sc_gather(codebook, idx, zeros_k):
    info = plsc.get_sparse_core_info()
    nc = info.num_cores
    nw = nc * info.num_subcores
    bpw = _B // nw
    mesh = plsc.VectorSubcoreMesh(core_axis_name="c", subcore_axis_name="s")

    @functools.partial(
        pl.kernel, mesh=mesh,
        out_type=[
            jax.ShapeDtypeStruct((_B, _D), jnp.float32),
            jax.ShapeDtypeStruct((nc, _K), jnp.float32),
        ],
        scratch_types=[
            pltpu.VMEM((bpw,), jnp.int32),
            pltpu.VMEM((bpw, _D), jnp.float32),
            pltpu.VMEM((bpw,), jnp.float32),
            pltpu.VMEM_SHARED((_K,), jnp.float32),
            pltpu.SemaphoreType.DMA,
        ],
    )
    def _k(cb_hbm, idx_hbm, z_hbm, out_hbm, cnt_hbm, idx_v, rows_v, ones_v,
           cnt_sh, sem):
        cid = lax.axis_index("c")
        sid = lax.axis_index("s")
        wid = sid * nc + cid
        base = wid * bpw
        pltpu.sync_copy(idx_hbm.at[pl.ds(base, bpw)], idx_v)
        cp = pltpu.async_copy(cb_hbm.at[idx_v], rows_v, sem)

        @pl.when(sid == 0)
        def _():
            pltpu.sync_copy(z_hbm.at[pl.ds(0, _K)], cnt_sh)
        one = jnp.full((16,), 1.0, jnp.float32)
        for q in range(bpw // 16):
            ones_v[pl.ds(q * 16, 16)] = one
        plsc.subcore_barrier()
        pltpu.sync_copy(ones_v, cnt_sh.at[idx_v], add=True)
        plsc.subcore_barrier()

        @pl.when(sid == 0)
        def _():
            pltpu.sync_copy(cnt_sh, cnt_hbm.at[cid])
        cp.wait()
        pltpu.sync_copy(rows_v, out_hbm.at[pl.ds(base, bpw)])

    return _k(codebook, idx, zeros_k)


def _dec2_body(d_ref, s_ref, v_ref, g_ref, bb_ref, w2_ref, b2_ref, cnt_ref,
               ds_ref, p_ref, sc_ref):
    i = pl.program_id(0)
    mu = s_ref[...] / float(_B)
    var = v_ref[...] / float(_B)
    dn = (d_ref[...] - mu) / jnp.sqrt(var + _EPS) * g_ref[...] + bb_ref[...]
    a = jnp.maximum(dn, 0.0)
    p_ref[...] = jnp.dot(a, w2_ref[...], preferred_element_type=jnp.float32) + b2_ref[...]

    @pl.when(i == 0)
    def _():
        lsum = ds_ref[0, 0] / float(_B * _D)
        vq = lsum + 0.25 * lsum
        avg = jnp.sum(cnt_ref[...], axis=0, keepdims=True) / float(_B)
        perp = jnp.exp(-jnp.sum(avg * jnp.log(avg + 1e-10)))
        lane = lax.broadcasted_iota(jnp.int32, (1, 128), 1)
        sc_ref[...] = jnp.where(lane == 0, vq, jnp.where(lane == 1, perp, 0.0))


def _dec2(d1, colsum, varsum, g, b, w2, b2, counts, dsum):
    return pl.pallas_call(
        _dec2_body,
        grid=(_NBLK,),
        in_specs=[
            pl.BlockSpec((_BB, _H), lambda i: (i, 0)),
            pl.BlockSpec((1, _H), lambda i: (0, 0)),
            pl.BlockSpec((1, _H), lambda i: (0, 0)),
            pl.BlockSpec((1, _H), lambda i: (0, 0)),
            pl.BlockSpec((1, _H), lambda i: (0, 0)),
            pl.BlockSpec((_H, _IN), lambda i: (0, 0)),
            pl.BlockSpec((1, _IN), lambda i: (0, 0)),
            pl.BlockSpec((2, _K), lambda i: (0, 0)),
            pl.BlockSpec((1, 128), lambda i: (0, 0)),
        ],
        out_specs=[
            pl.BlockSpec((_BB, _IN), lambda i: (i, 0)),
            pl.BlockSpec((1, 128), lambda i: (0, 0)),
        ],
        out_shape=[
            jax.ShapeDtypeStruct((_B, _IN), jnp.float32),
            jax.ShapeDtypeStruct((1, 128), jnp.float32),
        ],
    )(d1, colsum, varsum, g.reshape(1, _H), b.reshape(1, _H), w2,
      b2.reshape(1, _IN), counts, dsum)


def kernel(x, enc_W1, enc_b1, bn1_g, bn1_b, enc_W2, enc_b2, codebook,
           dec_W1, dec_b1, bn2_g, bn2_b, dec_W2, dec_b2):
    h, hsum = _mm_bias(x, enc_W1, enc_b1, _IN, _H)
    hvar = _varsum(h, hsum, _H)
    idx3, dsum = _enc2_vq(h, hsum, hvar, bn1_g, bn1_b, enc_W2, enc_b2,
                          codebook)
    idx = idx3.reshape(_B)
    quantized, counts = _sc_gather(codebook, idx, jnp.zeros((_K,), jnp.float32))
    d1, dsum1 = _mm_bias(quantized, dec_W1, dec_b1, _D, _H)
    dvar = _varsum(d1, dsum1, _H)
    predicted, scal = _dec2(d1, dsum1, dvar, bn2_g, bn2_b, dec_W2, dec_b2,
                            counts, dsum)
    return (scal[0, 0], predicted, scal[0, 1], idx[:, None])


# speedup vs baseline: 1.0542x; 1.0542x over previous
import functools

import jax
import jax.numpy as jnp
from jax import lax
from jax.experimental import pallas as pl
from jax.experimental.pallas import tpu as pltpu
from jax.experimental.pallas import tpu_sc as plsc

_B = 4096
_IN = 1000
_H = 1000
_D = 256
_K = 8192
_EPS = 1e-5
_BB = 512
_NBLK = _B // _BB
_KC = 2048
_NKC = _K // _KC
_KS = 2048


def _mm_bias_body(x_ref, w_ref, b_ref, o_ref, s_ref):
    i = pl.program_id(0)
    h = jnp.dot(x_ref[...], w_ref[...], preferred_element_type=jnp.float32)
    h = h + b_ref[...]
    o_ref[...] = h
    part = jnp.sum(h, axis=0, keepdims=True)

    @pl.when(i == 0)
    def _():
        s_ref[...] = part

    @pl.when(i > 0)
    def _():
        s_ref[...] = s_ref[...] + part


def _mm_bias(x, w, b, n_in, n_out):
    return pl.pallas_call(
        _mm_bias_body,
        grid=(_NBLK,),
        in_specs=[
            pl.BlockSpec((_BB, n_in), lambda i: (i, 0)),
            pl.BlockSpec((n_in, n_out), lambda i: (0, 0)),
            pl.BlockSpec((1, n_out), lambda i: (0, 0)),
        ],
        out_specs=[
            pl.BlockSpec((_BB, n_out), lambda i: (i, 0)),
            pl.BlockSpec((1, n_out), lambda i: (0, 0)),
        ],
        out_shape=[
            jax.ShapeDtypeStruct((_B, n_out), jnp.float32),
            jax.ShapeDtypeStruct((1, n_out), jnp.float32),
        ],
    )(x, w, b.reshape(1, n_out))


def _varsum_body(h_ref, s_ref, v_ref):
    i = pl.program_id(0)
    mu = s_ref[...] / float(_B)
    dev = h_ref[...] - mu
    part = jnp.sum(dev * dev, axis=0, keepdims=True)

    @pl.when(i == 0)
    def _():
        v_ref[...] = part

    @pl.when(i > 0)
    def _():
        v_ref[...] = v_ref[...] + part


def _varsum(h, colsum, n_out):
    return pl.pallas_call(
        _varsum_body,
        grid=(_NBLK,),
        in_specs=[
            pl.BlockSpec((_BB, n_out), lambda i: (i, 0)),
            pl.BlockSpec((1, n_out), lambda i: (0, 0)),
        ],
        out_specs=pl.BlockSpec((1, n_out), lambda i: (0, 0)),
        out_shape=jax.ShapeDtypeStruct((1, n_out), jnp.float32),
    )(h, colsum)


def _enc2_vq_body(h_ref, s_ref, v_ref, g_ref, bb_ref, w2_ref, b2_ref, cb_ref,
                  idx_ref, ds_ref):
    i = pl.program_id(0)
    mu = s_ref[...] / float(_B)
    var = v_ref[...] / float(_B)
    hn = (h_ref[...] - mu) / jnp.sqrt(var + _EPS) * g_ref[...] + bb_ref[...]
    a = jnp.maximum(hn, 0.0)
    z = jnp.dot(a, w2_ref[...], preferred_element_type=jnp.float32) + b2_ref[...]

    z2 = jnp.sum(z * z, axis=1, keepdims=True)
    zz = z + z
    best_d = jnp.full((_BB,), jnp.inf, jnp.float32)
    best_i = jnp.zeros((_BB,), jnp.int32)
    iot = lax.broadcasted_iota(jnp.int32, (_BB, _KS), 1)
    for c in range(_NKC):
        cc = cb_ref[pl.ds(c * _KC, _KC), :]
        c2 = jnp.sum(cc * cc, axis=1)[None, :]
        zc2 = lax.dot_general(zz, cc, (((1,), (1,)), ((), ())),
                              preferred_element_type=jnp.float32)
        for t in range(_KC // _KS):
            lo = t * _KS
            d = (z2 + c2[:, lo:lo + _KS]) - zc2[:, lo:lo + _KS]
            tmin = jnp.min(d, axis=1)
            targ = jnp.min(jnp.where(d == tmin[:, None], iot, _KS), axis=1)
            upd = tmin < best_d
            best_i = jnp.where(upd, targ + (c * _KC + lo), best_i)
            best_d = jnp.where(upd, tmin, best_d)

    idx_ref[...] = best_i.reshape(1, 1, _BB)

    lane = lax.broadcasted_iota(jnp.int32, (1, 128), 1)
    dpart = jnp.where(lane == 0, jnp.sum(best_d), 0.0)

    @pl.when(i == 0)
    def _():
        ds_ref[...] = dpart

    @pl.when(i > 0)
    def _():
        ds_ref[...] = ds_ref[...] + dpart


def _enc2_vq(h, colsum, varsum, g, b, w2, b2, codebook):
    return pl.pallas_call(
        _enc2_vq_body,
        grid=(_NBLK,),
        in_specs=[
            pl.BlockSpec((_BB, _H), lambda i: (i, 0)),
            pl.BlockSpec((1, _H), lambda i: (0, 0)),
            pl.BlockSpec((1, _H), lambda i: (0, 0)),
            pl.BlockSpec((1, _H), lambda i: (0, 0)),
            pl.BlockSpec((1, _H), lambda i: (0, 0)),
            pl.BlockSpec((_H, _D), lambda i: (0, 0)),
            pl.BlockSpec((1, _D), lambda i: (0, 0)),
            pl.BlockSpec((_K, _D), lambda i: (0, 0)),
        ],
        out_specs=[
            pl.BlockSpec((1, 1, _BB), lambda i: (i, 0, 0)),
            pl.BlockSpec((1, 128), lambda i: (0, 0)),
        ],
        out_shape=[
            jax.ShapeDtypeStruct((_NBLK, 1, _BB), jnp.int32),
            jax.ShapeDtypeStruct((1, 128), jnp.float32),
        ],
    )(h, colsum, varsum, g.reshape(1, _H), b.reshape(1, _H), w2,
      b2.reshape(1, _D), codebook)


def _sc_gather(codebook, idx, zeros_k):
    info = plsc.get_sparse_core_info()
    nc = info.num_cores
    nw = nc * info.num_subcores
    bpw = _B // nw
    mesh = plsc.VectorSubcoreMesh(core_axis_name="c", subcore_axis_name="s")

    @functools.partial(
        pl.kernel, mesh=mesh,
        out_type=[
            jax.ShapeDtypeStruct((_B, _D), jnp.float32),
            jax.ShapeDtypeStruct((nc, _K), jnp.float32),
        ],
        scratch_types=[
            pltpu.VMEM((bpw,), jnp.int32),
            pltpu.VMEM((bpw, _D), jnp.float32),
            pltpu.VMEM((bpw,), jnp.float32),
            pltpu.VMEM_SHARED((_K,), jnp.float32),
            pltpu.SemaphoreType.DMA,
        ],
    )
    def _k(cb_hbm, idx_hbm, z_hbm, out_hbm, cnt_hbm, idx_v, rows_v, ones_v,
           cnt_sh, sem):
        cid = lax.axis_index("c")
        sid = lax.axis_index("s")
        wid = sid * nc + cid
        base = wid * bpw
        pltpu.sync_copy(idx_hbm.at[pl.ds(base, bpw)], idx_v)
        cp = pltpu.async_copy(cb_hbm.at[idx_v], rows_v, sem)

        @pl.when(sid == 0)
        def _():
            pltpu.sync_copy(z_hbm.at[pl.ds(0, _K)], cnt_sh)
        one = jnp.full((16,), 1.0, jnp.float32)
        for q in range(bpw // 16):
            ones_v[pl.ds(q * 16, 16)] = one
        plsc.subcore_barrier()
        pltpu.sync_copy(ones_v, cnt_sh.at[idx_v], add=True)
        plsc.subcore_barrier()

        @pl.when(sid == 0)
        def _():
            pltpu.sync_copy(cnt_sh, cnt_hbm.at[cid])
        cp.wait()
        pltpu.sync_copy(rows_v, out_hbm.at[pl.ds(base, bpw)])

    return _k(codebook, idx, zeros_k)


def _dec2_body(d_ref, s_ref, v_ref, g_ref, bb_ref, w2_ref, b2_ref, cnt_ref,
               ds_ref, p_ref, sc_ref):
    i = pl.program_id(0)
    mu = s_ref[...] / float(_B)
    var = v_ref[...] / float(_B)
    dn = (d_ref[...] - mu) / jnp.sqrt(var + _EPS) * g_ref[...] + bb_ref[...]
    a = jnp.maximum(dn, 0.0)
    p_ref[...] = jnp.dot(a, w2_ref[...], preferred_element_type=jnp.float32) + b2_ref[...]

    @pl.when(i == 0)
    def _():
        lsum = ds_ref[0, 0] / float(_B * _D)
        vq = lsum + 0.25 * lsum
        avg = jnp.sum(cnt_ref[...], axis=0, keepdims=True) / float(_B)
        perp = jnp.exp(-jnp.sum(avg * jnp.log(avg + 1e-10)))
        lane = lax.broadcasted_iota(jnp.int32, (1, 128), 1)
        sc_ref[...] = jnp.where(lane == 0, vq, jnp.where(lane == 1, perp, 0.0))


def _dec2(d1, colsum, varsum, g, b, w2, b2, counts, dsum):
    return pl.pallas_call(
        _dec2_body,
        grid=(_NBLK,),
        in_specs=[
            pl.BlockSpec((_BB, _H), lambda i: (i, 0)),
            pl.BlockSpec((1, _H), lambda i: (0, 0)),
            pl.BlockSpec((1, _H), lambda i: (0, 0)),
            pl.BlockSpec((1, _H), lambda i: (0, 0)),
            pl.BlockSpec((1, _H), lambda i: (0, 0)),
            pl.BlockSpec((_H, _IN), lambda i: (0, 0)),
            pl.BlockSpec((1, _IN), lambda i: (0, 0)),
            pl.BlockSpec((2, _K), lambda i: (0, 0)),
            pl.BlockSpec((1, 128), lambda i: (0, 0)),
        ],
        out_specs=[
            pl.BlockSpec((_BB, _IN), lambda i: (i, 0)),
            pl.BlockSpec((1, 128), lambda i: (0, 0)),
        ],
        out_shape=[
            jax.ShapeDtypeStruct((_B, _IN), jnp.float32),
            jax.ShapeDtypeStruct((1, 128), jnp.float32),
        ],
    )(d1, colsum, varsum, g.reshape(1, _H), b.reshape(1, _H), w2,
      b2.reshape(1, _IN), counts, dsum)


def kernel(x, enc_W1, enc_b1, bn1_g, bn1_b, enc_W2, enc_b2, codebook,
           dec_W1, dec_b1, bn2_g, bn2_b, dec_W2, dec_b2):
    h, hsum = _mm_bias(x, enc_W1, enc_b1, _IN, _H)
    hvar = _varsum(h, hsum, _H)
    idx3, dsum = _enc2_vq(h, hsum, hvar, bn1_g, bn1_b, enc_W2, enc_b2,
                          codebook)
    idx = idx3.reshape(_B)
    quantized, counts = _sc_gather(codebook, idx, jnp.zeros((_K,), jnp.float32))
    d1, dsum1 = _mm_bias(quantized, dec_W1, dec_b1, _D, _H)
    dvar = _varsum(d1, dsum1, _H)
    predicted, scal = _dec2(d1, dsum1, dvar, bn2_g, bn2_b, dec_W2, dec_b2,
                            counts, dsum)
    return (scal[0, 0], predicted, scal[0, 1], idx[:, None])


# speedup vs baseline: 1.1950x; 1.1335x over previous
import functools

import jax
import jax.numpy as jnp
from jax import lax
from jax.experimental import pallas as pl
from jax.experimental.pallas import tpu as pltpu
from jax.experimental.pallas import tpu_sc as plsc

_B = 4096
_IN = 1000
_H = 1000
_D = 256
_K = 8192
_EPS = 1e-5
_BB = 512
_NBLK = _B // _BB
_KC = 2048
_NKC = _K // _KC


def _enc_body(x_ref, w1_ref, b1_ref, g_ref, bb_ref, w2_ref, b2_ref, cb_ref,
              idx_ref, sum_ref, var_ref, ds_ref, h_scr):
    p = pl.program_id(0)
    i = pl.program_id(1)

    @pl.when(p == 0)
    def _():
        h = jnp.dot(x_ref[...], w1_ref[...], preferred_element_type=jnp.float32)
        h = h + b1_ref[...]
        h_scr[pl.ds(i * _BB, _BB), :] = h
        part = jnp.sum(h, axis=0, keepdims=True)

        @pl.when(i == 0)
        def _():
            sum_ref[...] = part

        @pl.when(i > 0)
        def _():
            sum_ref[...] = sum_ref[...] + part

    @pl.when(p == 1)
    def _():
        mu = sum_ref[...] / float(_B)
        dev = h_scr[pl.ds(i * _BB, _BB), :] - mu
        part = jnp.sum(dev * dev, axis=0, keepdims=True)

        @pl.when(i == 0)
        def _():
            var_ref[...] = part

        @pl.when(i > 0)
        def _():
            var_ref[...] = var_ref[...] + part

    @pl.when(p == 2)
    def _():
        mu = sum_ref[...] / float(_B)
        var = var_ref[...] / float(_B)
        hn = (h_scr[pl.ds(i * _BB, _BB), :] - mu) / jnp.sqrt(var + _EPS)
        hn = hn * g_ref[...] + bb_ref[...]
        a = jnp.maximum(hn, 0.0)
        z = jnp.dot(a, w2_ref[...], preferred_element_type=jnp.float32)
        z = z + b2_ref[...]

        z2 = jnp.sum(z * z, axis=1, keepdims=True)
        zz = z + z
        best_d = jnp.full((_BB,), jnp.inf, jnp.float32)
        best_i = jnp.zeros((_BB,), jnp.int32)
        iot = lax.broadcasted_iota(jnp.int32, (_BB, _KC), 1)
        for c in range(_NKC):
            cc = cb_ref[pl.ds(c * _KC, _KC), :]
            c2 = jnp.sum(cc * cc, axis=1)[None, :]
            zc2 = lax.dot_general(zz, cc, (((1,), (1,)), ((), ())),
                                  preferred_element_type=jnp.float32)
            d = (z2 + c2) - zc2
            cmin = jnp.min(d, axis=1)
            carg = jnp.min(jnp.where(d == cmin[:, None], iot, _KC), axis=1)
            upd = cmin < best_d
            best_i = jnp.where(upd, carg + c * _KC, best_i)
            best_d = jnp.where(upd, cmin, best_d)

        idx_ref[...] = best_i.reshape(1, 1, _BB)

        lane = lax.broadcasted_iota(jnp.int32, (1, 128), 1)
        dpart = jnp.where(lane == 0, jnp.sum(best_d), 0.0)

        @pl.when(i == 0)
        def _():
            ds_ref[...] = dpart

        @pl.when(i > 0)
        def _():
            ds_ref[...] = ds_ref[...] + dpart


def _enc_mega(x, w1, b1, g, b, w2, b2, codebook):
    return pl.pallas_call(
        _enc_body,
        grid=(3, _NBLK),
        in_specs=[
            pl.BlockSpec((_BB, _IN), lambda p, i: (jnp.where(p == 0, i, 0), 0)),
            pl.BlockSpec((_IN, _H), lambda p, i: (0, 0)),
            pl.BlockSpec((1, _H), lambda p, i: (0, 0)),
            pl.BlockSpec((1, _H), lambda p, i: (0, 0)),
            pl.BlockSpec((1, _H), lambda p, i: (0, 0)),
            pl.BlockSpec((_H, _D), lambda p, i: (0, 0)),
            pl.BlockSpec((1, _D), lambda p, i: (0, 0)),
            pl.BlockSpec((_K, _D), lambda p, i: (0, 0)),
        ],
        out_specs=[
            pl.BlockSpec((1, 1, _BB), lambda p, i: (i, 0, 0)),
            pl.BlockSpec((1, _H), lambda p, i: (0, 0)),
            pl.BlockSpec((1, _H), lambda p, i: (0, 0)),
            pl.BlockSpec((1, 128), lambda p, i: (0, 0)),
        ],
        out_shape=[
            jax.ShapeDtypeStruct((_NBLK, 1, _BB), jnp.int32),
            jax.ShapeDtypeStruct((1, _H), jnp.float32),
            jax.ShapeDtypeStruct((1, _H), jnp.float32),
            jax.ShapeDtypeStruct((1, 128), jnp.float32),
        ],
        scratch_shapes=[pltpu.VMEM((_B, _H), jnp.float32)],
    )(x, w1, b1.reshape(1, _H), g.reshape(1, _H), b.reshape(1, _H), w2,
      b2.reshape(1, _D), codebook)


def _sc_gather(codebook, idx, zeros_k):
    info = plsc.get_sparse_core_info()
    nc = info.num_cores
    nw = nc * info.num_subcores
    bpw = _B // nw
    mesh = plsc.VectorSubcoreMesh(core_axis_name="c", subcore_axis_name="s")

    @functools.partial(
        pl.kernel, mesh=mesh,
        out_type=[
            jax.ShapeDtypeStruct((_B, _D), jnp.float32),
            jax.ShapeDtypeStruct((nc, _K), jnp.float32),
        ],
        scratch_types=[
            pltpu.VMEM((bpw,), jnp.int32),
            pltpu.VMEM((bpw, _D), jnp.float32),
            pltpu.VMEM((bpw,), jnp.float32),
            pltpu.VMEM_SHARED((_K,), jnp.float32),
            pltpu.SemaphoreType.DMA,
        ],
    )
    def _k(cb_hbm, idx_hbm, z_hbm, out_hbm, cnt_hbm, idx_v, rows_v, ones_v,
           cnt_sh, sem):
        cid = lax.axis_index("c")
        sid = lax.axis_index("s")
        wid = sid * nc + cid
        base = wid * bpw
        pltpu.sync_copy(idx_hbm.at[pl.ds(base, bpw)], idx_v)
        cp = pltpu.async_copy(cb_hbm.at[idx_v], rows_v, sem)

        @pl.when(sid == 0)
        def _():
            pltpu.sync_copy(z_hbm.at[pl.ds(0, _K)], cnt_sh)
        one = jnp.full((16,), 1.0, jnp.float32)
        for q in range(bpw // 16):
            ones_v[pl.ds(q * 16, 16)] = one
        plsc.subcore_barrier()
        pltpu.sync_copy(ones_v, cnt_sh.at[idx_v], add=True)
        plsc.subcore_barrier()

        @pl.when(sid == 0)
        def _():
            pltpu.sync_copy(cnt_sh, cnt_hbm.at[cid])
        cp.wait()
        pltpu.sync_copy(rows_v, out_hbm.at[pl.ds(base, bpw)])

    return _k(codebook, idx, zeros_k)


def _dec_body(q_ref, w1_ref, b1_ref, g_ref, bb_ref, w2_ref, b2_ref, cnt_ref,
              ds_ref, p_ref, sc_ref, sum_ref, var_ref, d_scr):
    p = pl.program_id(0)
    i = pl.program_id(1)

    @pl.when(p == 0)
    def _():
        d1 = jnp.dot(q_ref[...], w1_ref[...], preferred_element_type=jnp.float32)
        d1 = d1 + b1_ref[...]
        d_scr[pl.ds(i * _BB, _BB), :] = d1
        part = jnp.sum(d1, axis=0, keepdims=True)

        @pl.when(i == 0)
        def _():
            sum_ref[...] = part

        @pl.when(i > 0)
        def _():
            sum_ref[...] = sum_ref[...] + part

    @pl.when(p == 1)
    def _():
        mu = sum_ref[...] / float(_B)
        dev = d_scr[pl.ds(i * _BB, _BB), :] - mu
        part = jnp.sum(dev * dev, axis=0, keepdims=True)

        @pl.when(i == 0)
        def _():
            var_ref[...] = part

        @pl.when(i > 0)
        def _():
            var_ref[...] = var_ref[...] + part

    @pl.when(p == 2)
    def _():
        mu = sum_ref[...] / float(_B)
        var = var_ref[...] / float(_B)
        dn = (d_scr[pl.ds(i * _BB, _BB), :] - mu) / jnp.sqrt(var + _EPS)
        dn = dn * g_ref[...] + bb_ref[...]
        a = jnp.maximum(dn, 0.0)
        out = jnp.dot(a, w2_ref[...], preferred_element_type=jnp.float32)
        p_ref[...] = out + b2_ref[...]

        @pl.when(i == 0)
        def _():
            lsum = ds_ref[0, 0] / float(_B * _D)
            vq = lsum + 0.25 * lsum
            avg = jnp.sum(cnt_ref[...], axis=0, keepdims=True) / float(_B)
            perp = jnp.exp(-jnp.sum(avg * jnp.log(avg + 1e-10)))
            lane = lax.broadcasted_iota(jnp.int32, (1, 128), 1)
            sc_ref[...] = jnp.where(lane == 0, vq,
                                    jnp.where(lane == 1, perp, 0.0))


def _dec_mega(q, w1, b1, g, b, w2, b2, counts, dsum):
    return pl.pallas_call(
        _dec_body,
        grid=(3, _NBLK),
        in_specs=[
            pl.BlockSpec((_BB, _D), lambda p, i: (jnp.where(p == 0, i, 0), 0)),
            pl.BlockSpec((_D, _H), lambda p, i: (0, 0)),
            pl.BlockSpec((1, _H), lambda p, i: (0, 0)),
            pl.BlockSpec((1, _H), lambda p, i: (0, 0)),
            pl.BlockSpec((1, _H), lambda p, i: (0, 0)),
            pl.BlockSpec((_H, _IN), lambda p, i: (0, 0)),
            pl.BlockSpec((1, _IN), lambda p, i: (0, 0)),
            pl.BlockSpec((2, _K), lambda p, i: (0, 0)),
            pl.BlockSpec((1, 128), lambda p, i: (0, 0)),
        ],
        out_specs=[
            pl.BlockSpec((_BB, _IN), lambda p, i: (jnp.where(p == 2, i, 0), 0)),
            pl.BlockSpec((1, 128), lambda p, i: (0, 0)),
        ],
        out_shape=[
            jax.ShapeDtypeStruct((_B, _IN), jnp.float32),
            jax.ShapeDtypeStruct((1, 128), jnp.float32),
        ],
        scratch_shapes=[
            pltpu.VMEM((1, _H), jnp.float32),
            pltpu.VMEM((1, _H), jnp.float32),
            pltpu.VMEM((_B, _H), jnp.float32),
        ],
    )(q, w1, b1.reshape(1, _H), g.reshape(1, _H), b.reshape(1, _H), w2,
      b2.reshape(1, _IN), counts, dsum)


def kernel(x, enc_W1, enc_b1, bn1_g, bn1_b, enc_W2, enc_b2, codebook,
           dec_W1, dec_b1, bn2_g, bn2_b, dec_W2, dec_b2):
    idx3, hsum, hvar, dsum = _enc_mega(x, enc_W1, enc_b1, bn1_g, bn1_b,
                                       enc_W2, enc_b2, codebook)
    idx = idx3.reshape(_B)
    quantized, counts = _sc_gather(codebook, idx, jnp.zeros((_K,), jnp.float32))
    predicted, scal = _dec_mega(quantized, dec_W1, dec_b1, bn2_g, bn2_b,
                                dec_W2, dec_b2, counts, dsum)
    return (scal[0, 0], predicted, scal[0, 1], idx[:, None])
